# baseline (speedup 1.0000x reference)
import functools

import jax
import jax.numpy as jnp
from jax import lax
from jax.experimental import pallas as pl
from jax.experimental.pallas import tpu as pltpu
from jax.experimental.pallas import tpu_sc as plsc

N = 10000
E = 320000
D_IN = 128
D_H = 128
D_OUT = 64

NC = 2
NS = 16
NW = NC * NS

B = 128
CH = 80
E_PAD = NW * CH * B
N_PAD = 10240
RPT = N_PAD // NS
DUMMY = N

@functools.cache
def _mesh():
    return plsc.VectorSubcoreMesh(
        core_axis_name="c", subcore_axis_name="s", num_cores=NC, num_subcores=NS
    )


def _wid():
    return lax.axis_index("c") * NS + lax.axis_index("s")


@functools.cache
def _sc_degree():
    @functools.partial(
        pl.kernel,
        out_type=jax.ShapeDtypeStruct((NC, N_PAD, 16), jnp.float32),
        mesh=_mesh(),
        compiler_params=pltpu.CompilerParams(use_tc_tiling_on_sc=False),
        scratch_types=[
            pltpu.VMEM((CH, B), jnp.int32),
            pltpu.VMEM((B, 16), jnp.float32),
            pltpu.VMEM_SHARED((N_PAD, 16), jnp.float32),
        ],
    )
    def deg(dst_hbm, ones_hbm, zeros_hbm, out_hbm, dst_v, ones_v, acc):
        c = lax.axis_index("c")
        s = lax.axis_index("s")
        w = _wid()
        pltpu.sync_copy(dst_hbm.at[w], dst_v)
        pltpu.sync_copy(ones_hbm, ones_v)
        pltpu.sync_copy(
            zeros_hbm.at[pl.ds(s * RPT, RPT)], acc.at[pl.ds(s * RPT, RPT)]
        )
        plsc.subcore_barrier()

        def chunk(j, carry):
            pltpu.sync_copy(ones_v, acc.at[dst_v.at[j]], add=True)
            return carry

        lax.fori_loop(0, CH, chunk, 0)
        plsc.subcore_barrier()
        pltpu.sync_copy(
            acc.at[pl.ds(s * RPT, RPT)], out_hbm.at[c, pl.ds(s * RPT, RPT)]
        )

    return deg


@functools.cache
def _make_sc_aggregate(D, bb, ch, nbuf):
    @functools.partial(
        pl.kernel,
        out_type=jax.ShapeDtypeStruct((NC, N_PAD, D), jnp.float32),
        mesh=_mesh(),
        compiler_params=pltpu.CompilerParams(use_tc_tiling_on_sc=False),
        scratch_types=[
            pltpu.VMEM((ch, bb), jnp.int32),
            pltpu.VMEM((ch, bb), jnp.int32),
            [pltpu.VMEM((bb, D), jnp.float32)] * nbuf,
            pltpu.VMEM_SHARED((N_PAD, D), jnp.float32),
            [pltpu.SemaphoreType.DMA] * nbuf,
            [pltpu.SemaphoreType.DMA] * nbuf,
        ],
    )
    def agg(y_hbm, src_hbm, dst_hbm, zeros_hbm, out_hbm,
            src_v, dst_v, rows, acc, gsem, ssem):
        c = lax.axis_index("c")
        s = lax.axis_index("s")
        w = _wid()
        pltpu.sync_copy(src_hbm.at[w], src_v)
        pltpu.sync_copy(dst_hbm.at[w], dst_v)
        pltpu.sync_copy(
            zeros_hbm.at[pl.ds(s * RPT, RPT)], acc.at[pl.ds(s * RPT, RPT)]
        )
        plsc.subcore_barrier()

        def fire_gather(j, b):
            pltpu.async_copy(y_hbm.at[src_v.at[j]], rows[b], gsem[b])

        def wait_gather(j, b):
            pltpu.make_async_copy(y_hbm.at[src_v.at[j]], rows[b], gsem[b]).wait()

        def fire_scatter(j, b):
            pltpu.async_copy(rows[b], acc.at[dst_v.at[j]], ssem[b], add=True)

        def wait_scatter(j, b):
            pltpu.make_async_copy(rows[b], acc.at[dst_v.at[j]], ssem[b]).wait()

        T = ch // nbuf
        for b in range(nbuf):
            fire_gather(b, b)

        def group(t, carry):
            for b in range(nbuf):
                j = nbuf * t + b
                wait_gather(j, b)
                fire_scatter(j, b)
                wait_scatter(j, b)
                fire_gather(j + nbuf, b)
            return carry

        lax.fori_loop(0, T - 1, group, 0)
        for b in range(nbuf):
            j = nbuf * (T - 1) + b
            wait_gather(j, b)
            fire_scatter(j, b)
            wait_scatter(j, b)

        plsc.subcore_barrier()
        pltpu.sync_copy(
            acc.at[pl.ds(s * RPT, RPT)], out_hbm.at[c, pl.ds(s * RPT, RPT)]
        )

    return agg


R = 1024
_GRID = N_PAD // R


def _dinv_block(degp):
    deg = 1.0 + degp[0, :, 0:1] + degp[1, :, 0:1]
    return lax.rsqrt(deg)


def _tc1_body(x_ref, w1_ref, degp_ref, y1_ref):
    dinv = _dinv_block(degp_ref)
    xw = jnp.dot(x_ref[...], w1_ref[...], preferred_element_type=jnp.float32)
    y1_ref[...] = xw * dinv


def _tc2_body(s1_ref, y1_ref, degp_ref, b1_ref, w2_ref, y2_ref):
    dinv = _dinv_block(degp_ref)
    pre = dinv * (s1_ref[0] + s1_ref[1] + y1_ref[...]) + b1_ref[...]
    h = jnp.maximum(pre, 0.0)
    y2_ref[...] = jnp.dot(h, w2_ref[...], preferred_element_type=jnp.float32) * dinv


def _tc3_body(s2_ref, y2_ref, degp_ref, b2_ref, out_ref):
    dinv = _dinv_block(degp_ref)
    out_ref[...] = dinv * (s2_ref[0] + s2_ref[1] + y2_ref[...]) + b2_ref[...]


def _row_spec(d):
    return pl.BlockSpec((R, d), lambda i: (i, 0))


def _part_spec(d):
    return pl.BlockSpec((NC, R, d), lambda i: (0, i, 0))


_DEGP_SPEC = pl.BlockSpec((NC, R, 16), lambda i: (0, i, 0))


def _full_spec(shape):
    return pl.BlockSpec(shape, lambda i: tuple(0 for _ in shape))


_tc1 = pl.pallas_call(
    _tc1_body,
    grid=(_GRID,),
    in_specs=[_row_spec(D_IN), _full_spec((D_IN, D_H)), _DEGP_SPEC],
    out_specs=_row_spec(D_H),
    out_shape=jax.ShapeDtypeStruct((N_PAD, D_H), jnp.float32),
)

_tc2 = pl.pallas_call(
    _tc2_body,
    grid=(_GRID,),
    in_specs=[
        _part_spec(D_H),
        _row_spec(D_H),
        _DEGP_SPEC,
        _full_spec((1, D_H)),
        _full_spec((D_H, D_OUT)),
    ],
    out_specs=_row_spec(D_OUT),
    out_shape=jax.ShapeDtypeStruct((N_PAD, D_OUT), jnp.float32),
)

_tc3 = pl.pallas_call(
    _tc3_body,
    grid=(_GRID,),
    in_specs=[_part_spec(D_OUT), _row_spec(D_OUT), _DEGP_SPEC, _full_spec((1, D_OUT))],
    out_specs=_row_spec(D_OUT),
    out_shape=jax.ShapeDtypeStruct((N_PAD, D_OUT), jnp.float32),
)


@jax.jit
def kernel(label_features, edge_index, W1, b1, W2, b2):
    xp = jnp.zeros((N_PAD, D_IN), jnp.float32).at[:N].set(label_features)
    pad = jnp.full((E_PAD - E,), DUMMY, jnp.int32)
    src_flat = jnp.concatenate([edge_index[0], pad])
    dst_flat = jnp.concatenate([edge_index[1], pad])
    src128, dst128 = src_flat.reshape(NW, CH, B), dst_flat.reshape(NW, CH, B)
    src64 = src_flat.reshape(NW, 2 * CH, B // 2)
    dst64 = dst_flat.reshape(NW, 2 * CH, B // 2)
    ones16 = jnp.ones((B, 16), jnp.float32)
    z16 = jnp.zeros((N_PAD, 16), jnp.float32)
    zh = jnp.zeros((N_PAD, D_H), jnp.float32)
    zo = jnp.zeros((N_PAD, D_OUT), jnp.float32)

    degp = _sc_degree()(dst128, ones16, z16)

    y1 = _tc1(xp, W1, degp)
    s1p = _make_sc_aggregate(D_H, B // 2, 2 * CH, 2)(y1, src64, dst64, zh)
    y2 = _tc2(s1p, y1, degp, b1.reshape(1, D_H), W2)
    s2p = _make_sc_aggregate(D_OUT, B, CH, 4)(y2, src128, dst128, zo)
    out = _tc3(s2p, y2, degp, b2.reshape(1, D_OUT))
    return out[:N]


# speedup vs baseline: 1.5419x; 1.5419x over previous
import functools

import jax
import jax.numpy as jnp
from jax import lax
from jax.experimental import pallas as pl
from jax.experimental.pallas import tpu as pltpu
from jax.experimental.pallas import tpu_sc as plsc

N = 10000
E = 320000
D_IN = 128
D_H = 128
D_OUT = 64

NC = 2
NS = 16
NW = NC * NS

B = 128
SLOW_CORE = 0
CH_S = 56
CH_F = 101
CH_MAX = max(CH_S, CH_F)
E_S = NS * CH_S * B
E_F = NS * CH_F * B
N_PAD = 10240
RPT = N_PAD // NS
DUMMY = N

@functools.cache
def _mesh():
    return plsc.VectorSubcoreMesh(
        core_axis_name="c", subcore_axis_name="s", num_cores=NC, num_subcores=NS
    )


def _wid():
    return lax.axis_index("c") * NS + lax.axis_index("s")


@functools.cache
def _sc_degree():
    @functools.partial(
        pl.kernel,
        out_type=jax.ShapeDtypeStruct((NC, N_PAD, 16), jnp.float32),
        mesh=_mesh(),
        compiler_params=pltpu.CompilerParams(use_tc_tiling_on_sc=False),
        scratch_types=[
            pltpu.VMEM((CH_MAX, B), jnp.int32),
            pltpu.VMEM((B, 16), jnp.float32),
            pltpu.VMEM_SHARED((N_PAD, 16), jnp.float32),
        ],
    )
    def deg(dst_hbm, ones_hbm, zeros_hbm, out_hbm, dst_v, ones_v, acc):
        c = lax.axis_index("c")
        s = lax.axis_index("s")
        w = _wid()
        t_c = jnp.where(c == SLOW_CORE, CH_S, CH_F)
        pltpu.sync_copy(dst_hbm.at[w], dst_v)
        pltpu.sync_copy(ones_hbm, ones_v)
        pltpu.sync_copy(
            zeros_hbm.at[pl.ds(s * RPT, RPT)], acc.at[pl.ds(s * RPT, RPT)]
        )
        plsc.subcore_barrier()

        def chunk(j, carry):
            pltpu.sync_copy(ones_v, acc.at[dst_v.at[j]], add=True)
            return carry

        lax.fori_loop(0, t_c, chunk, 0)
        plsc.subcore_barrier()
        pltpu.sync_copy(
            acc.at[pl.ds(s * RPT, RPT)], out_hbm.at[c, pl.ds(s * RPT, RPT)]
        )

    return deg


@functools.cache
def _make_sc_aggregate(D):
    @functools.partial(
        pl.kernel,
        out_type=jax.ShapeDtypeStruct((NC, N_PAD, D), jnp.float32),
        mesh=_mesh(),
        compiler_params=pltpu.CompilerParams(use_tc_tiling_on_sc=False),
        scratch_types=[
            pltpu.VMEM((CH_MAX, B), jnp.int32),
            pltpu.VMEM((CH_MAX, B), jnp.int32),
            pltpu.VMEM((B, D), jnp.float32),
            pltpu.VMEM_SHARED((N_PAD, D), jnp.float32),
            pltpu.SemaphoreType.DMA,
        ],
    )
    def agg(y_hbm, src_hbm, dst_hbm, zeros_hbm, out_hbm,
            src_v, dst_v, rows_v, acc, sem):
        c = lax.axis_index("c")
        s = lax.axis_index("s")
        w = _wid()
        t_c = jnp.where(c == SLOW_CORE, CH_S, CH_F)
        pltpu.sync_copy(src_hbm.at[w], src_v)
        pltpu.sync_copy(dst_hbm.at[w], dst_v)
        pltpu.sync_copy(
            zeros_hbm.at[pl.ds(s * RPT, RPT)], acc.at[pl.ds(s * RPT, RPT)]
        )
        plsc.subcore_barrier()

        def chunk(j, carry):
            pltpu.async_copy(y_hbm.at[src_v.at[j]], rows_v, sem).wait()
            pltpu.sync_copy(rows_v, acc.at[dst_v.at[j]], add=True)
            return carry

        lax.fori_loop(0, t_c, chunk, 0)
        plsc.subcore_barrier()
        pltpu.sync_copy(
            acc.at[pl.ds(s * RPT, RPT)], out_hbm.at[c, pl.ds(s * RPT, RPT)]
        )

    return agg


R = 1024
_GRID = N_PAD // R


def _dinv_block(degp):
    deg = 1.0 + degp[0, :, 0:1] + degp[1, :, 0:1]
    return lax.rsqrt(deg)


def _tc1_body(x_ref, w1_ref, degp_ref, y1_ref):
    dinv = _dinv_block(degp_ref)
    xw = jnp.dot(x_ref[...], w1_ref[...], preferred_element_type=jnp.float32)
    y1_ref[...] = xw * dinv


def _tc2_body(s1_ref, y1_ref, degp_ref, b1_ref, w2_ref, y2_ref):
    dinv = _dinv_block(degp_ref)
    pre = dinv * (s1_ref[0] + s1_ref[1] + y1_ref[...]) + b1_ref[...]
    h = jnp.maximum(pre, 0.0)
    y2_ref[...] = jnp.dot(h, w2_ref[...], preferred_element_type=jnp.float32) * dinv


def _tc3_body(s2_ref, y2_ref, degp_ref, b2_ref, out_ref):
    dinv = _dinv_block(degp_ref)
    out_ref[...] = dinv * (s2_ref[0] + s2_ref[1] + y2_ref[...]) + b2_ref[...]


def _row_spec(d):
    return pl.BlockSpec((R, d), lambda i: (i, 0))


def _part_spec(d):
    return pl.BlockSpec((NC, R, d), lambda i: (0, i, 0))


_DEGP_SPEC = pl.BlockSpec((NC, R, 16), lambda i: (0, i, 0))


def _full_spec(shape):
    return pl.BlockSpec(shape, lambda i: tuple(0 for _ in shape))


_tc1 = pl.pallas_call(
    _tc1_body,
    grid=(_GRID,),
    in_specs=[_row_spec(D_IN), _full_spec((D_IN, D_H)), _DEGP_SPEC],
    out_specs=_row_spec(D_H),
    out_shape=jax.ShapeDtypeStruct((N_PAD, D_H), jnp.float32),
)

_tc2 = pl.pallas_call(
    _tc2_body,
    grid=(_GRID,),
    in_specs=[
        _part_spec(D_H),
        _row_spec(D_H),
        _DEGP_SPEC,
        _full_spec((1, D_H)),
        _full_spec((D_H, D_OUT)),
    ],
    out_specs=_row_spec(D_OUT),
    out_shape=jax.ShapeDtypeStruct((N_PAD, D_OUT), jnp.float32),
)

_tc3 = pl.pallas_call(
    _tc3_body,
    grid=(_GRID,),
    in_specs=[_part_spec(D_OUT), _row_spec(D_OUT), _DEGP_SPEC, _full_spec((1, D_OUT))],
    out_specs=_row_spec(D_OUT),
    out_shape=jax.ShapeDtypeStruct((N_PAD, D_OUT), jnp.float32),
)


def _edge_layout(idx):
    slow = idx[:E_S].reshape(NS, CH_S, B)
    slow = jnp.pad(slow, ((0, 0), (0, CH_MAX - CH_S), (0, 0)),
                   constant_values=DUMMY)
    rest = jnp.concatenate(
        [idx[E_S:], jnp.full((E_F - (E - E_S),), DUMMY, jnp.int32)]
    )
    fast = rest.reshape(NS, CH_F, B)
    blocks = [slow, fast] if SLOW_CORE == 0 else [fast, slow]
    return jnp.concatenate(blocks, axis=0)


@jax.jit
def kernel(label_features, edge_index, W1, b1, W2, b2):
    xp = jnp.zeros((N_PAD, D_IN), jnp.float32).at[:N].set(label_features)
    src = _edge_layout(edge_index[0])
    dst = _edge_layout(edge_index[1])
    ones16 = jnp.ones((B, 16), jnp.float32)
    z16 = jnp.zeros((N_PAD, 16), jnp.float32)
    zh = jnp.zeros((N_PAD, D_H), jnp.float32)
    zo = jnp.zeros((N_PAD, D_OUT), jnp.float32)

    degp = _sc_degree()(dst, ones16, z16)

    y1 = _tc1(xp, W1, degp)
    s1p = _make_sc_aggregate(D_H)(y1, src, dst, zh)
    y2 = _tc2(s1p, y1, degp, b1.reshape(1, D_H), W2)
    s2p = _make_sc_aggregate(D_OUT)(y2, src, dst, zo)
    out = _tc3(s2p, y2, degp, b2.reshape(1, D_OUT))
    return out[:N]
